# baseline (speedup 1.0000x reference)
import functools

import jax
import jax.numpy as jnp
from jax import lax
from jax.experimental import pallas as pl
from jax.experimental.pallas import tpu as pltpu
from jax.experimental.pallas import tpu_sc as plsc

_D = 64
_PB = 16384
_CHUNK = 128
_BW = 512
_SPLIT = 13


def _transform_body(x_ref, wt_ref, b_ref, o_ref):
    x = x_ref[...]
    for half in range(2):
        xt = x[:, half * _PB:(half + 1) * _PB].T
        y = jnp.dot(xt, wt_ref[...], preferred_element_type=jnp.float32)
        y = jnp.maximum(y + b_ref[...], 0.0)
        o_ref[:, half * _D:(half + 1) * _D] = y


def _tc_transform(table_t, wt, b2d):
    v, n = table_t.shape
    grid_n = (n + 2 * _PB - 1) // (2 * _PB)
    return pl.pallas_call(
        _transform_body,
        grid=(grid_n,),
        in_specs=[
            pl.BlockSpec((_D, 2 * _PB), lambda i: (0, i)),
            pl.BlockSpec((_D, _D), lambda i: (0, 0)),
            pl.BlockSpec((1, _D), lambda i: (0, 0)),
        ],
        out_specs=pl.BlockSpec((_PB, 2 * _D), lambda i: (i, 0)),
        out_shape=jax.ShapeDtypeStruct((grid_n * _PB, 2 * _D), jnp.float32),
    )(table_t, wt, b2d)


def _gather_body(n_pairs, table_hbm, idxt_hbm, out_hbm, idx_v, rows_v, sem):
    info = plsc.get_sparse_core_info()
    nc = info.num_cores
    wid = lax.axis_index("s") * nc + lax.axis_index("c")
    b0 = wid * _BW
    pltpu.sync_copy(idxt_hbm.at[:, pl.ds(b0, _BW)], idx_v)

    def pair(j, carry):
        copies = []
        for c in range(_BW // _CHUNK):
            src_e = idx_v.at[2 * j, pl.ds(c * _CHUNK, _CHUNK)]
            src_o = idx_v.at[2 * j + 1, pl.ds(c * _CHUNK, _CHUNK)]
            copies.append(pltpu.async_copy(
                table_hbm.at[src_e], rows_v.at[pl.ds(c * _CHUNK, _CHUNK)], sem))
            copies.append(pltpu.async_copy(
                table_hbm.at[src_o],
                rows_v.at[pl.ds(_BW + c * _CHUNK, _CHUNK)], sem))
        for cp in copies:
            cp.wait()
        pltpu.sync_copy(rows_v.at[pl.ds(0, _BW)],
                        out_hbm.at[j, pl.ds(b0, _BW), pl.ds(0, _D)])
        pltpu.sync_copy(rows_v.at[pl.ds(_BW, _BW)],
                        out_hbm.at[j, pl.ds(b0, _BW), pl.ds(_D, _D)])
        return carry

    lax.fori_loop(0, n_pairs, pair, 0)


def _sc_gather(table_lin, idxt):
    hist, batch = idxt.shape
    n_pairs = hist // 2
    mesh = plsc.VectorSubcoreMesh(core_axis_name="c", subcore_axis_name="s")
    kern = pl.kernel(
        functools.partial(_gather_body, n_pairs),
        mesh=mesh,
        compiler_params=pltpu.CompilerParams(use_tc_tiling_on_sc=False),
        out_type=jax.ShapeDtypeStruct((n_pairs, batch, 2 * _D), jnp.float32),
        scratch_types=[
            pltpu.VMEM((hist, _BW), jnp.int32),
            pltpu.VMEM((2 * _BW, _D), jnp.float32),
            pltpu.SemaphoreType.DMA,
        ],
    )
    return kern(table_lin, idxt)


def _unpack_body(x_ref, o_ref):
    xt = x_ref[0].T
    o_ref[0] = xt[:_D, :]
    o_ref[1] = xt[_D:, :]


def _unpack_alias_body(x_ref, y_ref, o_ref):
    del y_ref
    _unpack_body(x_ref, o_ref)


def _tc_unpack_first(emb3, hist, batch, bl):
    n_pairs = emb3.shape[0]
    return pl.pallas_call(
        _unpack_body,
        grid=(n_pairs, batch // bl),
        in_specs=[pl.BlockSpec((1, bl, 2 * _D), lambda j, i: (j, i, 0))],
        out_specs=pl.BlockSpec((2, _D, bl), lambda j, i: (j, 0, i)),
        out_shape=jax.ShapeDtypeStruct((hist, _D, batch), jnp.float32),
    )(emb3)


def _tc_unpack_second(emb3, y3, bl, j_off):
    n_pairs, batch, _ = emb3.shape
    return pl.pallas_call(
        _unpack_alias_body,
        grid=(n_pairs, batch // bl),
        in_specs=[
            pl.BlockSpec((1, bl, 2 * _D), lambda j, i: (j, i, 0)),
            pl.BlockSpec(memory_space=pltpu.MemorySpace.HBM),
        ],
        out_specs=pl.BlockSpec((2, _D, bl), lambda j, i, o=j_off: (j + o, 0, i)),
        out_shape=jax.ShapeDtypeStruct(y3.shape, jnp.float32),
        input_output_aliases={1: 0},
    )(emb3, y3)


def kernel(element, table, W, b):
    batch, hist = element.shape
    el = element.astype(jnp.int32)
    i = el // (2 * _PB)
    q = el % (2 * _PB)
    idx_phys = i * (2 * _PB) + 2 * (q % _PB) + q // _PB
    idxt = idx_phys.T
    p2 = _tc_transform(table.T, W.T, b.reshape(1, _D))
    p2v = p2.reshape(p2.shape[0] * 2, _D)
    emb3_a = _sc_gather(p2v, idxt[:2 * _SPLIT])
    emb3_b = _sc_gather(p2v, idxt[2 * _SPLIT:])
    y3 = _tc_unpack_first(emb3_a, hist, batch, batch)
    y3 = _tc_unpack_second(emb3_b, y3, batch, _SPLIT)
    return y3.transpose(2, 0, 1)


# speedup vs baseline: 1.0080x; 1.0080x over previous
import functools

import jax
import jax.numpy as jnp
from jax import lax
from jax.experimental import pallas as pl
from jax.experimental.pallas import tpu as pltpu
from jax.experimental.pallas import tpu_sc as plsc

_D = 64
_PB = 16384
_CHUNK = 128
_BW = 512
_SPLIT = 13


def _transform_body(x_ref, wt_ref, b_ref, o_ref):
    x = x_ref[...]
    for half in range(2):
        xt = x[:, half * _PB:(half + 1) * _PB].T
        y = jnp.dot(xt, wt_ref[...], preferred_element_type=jnp.float32)
        y = jnp.maximum(y + b_ref[...], 0.0)
        o_ref[:, half * _D:(half + 1) * _D] = y


def _tc_transform(table_t, wt, b2d):
    v, n = table_t.shape
    grid_n = (n + 2 * _PB - 1) // (2 * _PB)
    return pl.pallas_call(
        _transform_body,
        grid=(grid_n,),
        in_specs=[
            pl.BlockSpec((_D, 2 * _PB), lambda i: (0, i)),
            pl.BlockSpec((_D, _D), lambda i: (0, 0)),
            pl.BlockSpec((1, _D), lambda i: (0, 0)),
        ],
        out_specs=pl.BlockSpec((_PB, 2 * _D), lambda i: (i, 0)),
        out_shape=jax.ShapeDtypeStruct((grid_n * _PB, 2 * _D), jnp.float32),
    )(table_t, wt, b2d)


def _gather_body(n_pairs, table_hbm, idxt_hbm, out_hbm, idx_v, rows_v, sem):
    info = plsc.get_sparse_core_info()
    nc = info.num_cores
    wid = lax.axis_index("s") * nc + lax.axis_index("c")
    b0 = wid * _BW
    pltpu.sync_copy(idxt_hbm.at[:, pl.ds(b0, _BW)], idx_v)

    def pair(j, carry):
        copies = []
        for c in range(_BW // _CHUNK):
            src_e = idx_v.at[2 * j, pl.ds(c * _CHUNK, _CHUNK)]
            src_o = idx_v.at[2 * j + 1, pl.ds(c * _CHUNK, _CHUNK)]
            copies.append(pltpu.async_copy(
                table_hbm.at[src_e], rows_v.at[pl.ds(c * _CHUNK, _CHUNK)], sem))
            copies.append(pltpu.async_copy(
                table_hbm.at[src_o],
                rows_v.at[pl.ds(_BW + c * _CHUNK, _CHUNK)], sem))
        for cp in copies:
            cp.wait()
        pltpu.sync_copy(rows_v.at[pl.ds(0, _BW)],
                        out_hbm.at[j, pl.ds(b0, _BW), pl.ds(0, _D)])
        pltpu.sync_copy(rows_v.at[pl.ds(_BW, _BW)],
                        out_hbm.at[j, pl.ds(b0, _BW), pl.ds(_D, _D)])
        return carry

    lax.fori_loop(0, n_pairs, pair, 0)


def _sc_gather(table_lin, idxt):
    hist, batch = idxt.shape
    n_pairs = hist // 2
    mesh = plsc.VectorSubcoreMesh(core_axis_name="c", subcore_axis_name="s")
    kern = pl.kernel(
        functools.partial(_gather_body, n_pairs),
        mesh=mesh,
        compiler_params=pltpu.CompilerParams(use_tc_tiling_on_sc=False),
        out_type=jax.ShapeDtypeStruct((n_pairs, batch, 2 * _D), jnp.float32),
        scratch_types=[
            pltpu.VMEM((hist, _BW), jnp.int32),
            pltpu.VMEM((2 * _BW, _D), jnp.float32),
            pltpu.SemaphoreType.DMA,
        ],
    )
    return kern(table_lin, idxt)


def _unpack_body(x_ref, o_ref):
    xt = x_ref[0].T
    o_ref[0] = xt[:_D, :]
    o_ref[1] = xt[_D:, :]


def _unpack_alias_body(x_ref, y_ref, o_ref):
    del y_ref
    _unpack_body(x_ref, o_ref)


def _tc_unpack_first(emb3, hist, batch, bl):
    n_pairs = emb3.shape[0]
    return pl.pallas_call(
        _unpack_body,
        grid=(n_pairs, batch // bl),
        in_specs=[pl.BlockSpec((1, bl, 2 * _D), lambda j, i: (j, i, 0))],
        out_specs=pl.BlockSpec((2, _D, bl), lambda j, i: (j, 0, i)),
        out_shape=jax.ShapeDtypeStruct((hist, _D, batch), jnp.float32),
    )(emb3)


def _tc_unpack_second(emb3, y3, bl, j_off):
    n_pairs, batch, _ = emb3.shape
    return pl.pallas_call(
        _unpack_alias_body,
        grid=(n_pairs, batch // bl),
        in_specs=[
            pl.BlockSpec((1, bl, 2 * _D), lambda j, i: (j, i, 0)),
            pl.BlockSpec(memory_space=pltpu.MemorySpace.HBM),
        ],
        out_specs=pl.BlockSpec((2, _D, bl), lambda j, i, o=j_off: (j + o, 0, i)),
        out_shape=jax.ShapeDtypeStruct(y3.shape, jnp.float32),
        input_output_aliases={1: 0},
    )(emb3, y3)


def kernel(element, table, W, b):
    batch, hist = element.shape
    el = element.astype(jnp.int32)
    i = el // (2 * _PB)
    q = el % (2 * _PB)
    idx_phys = i * (2 * _PB) + 2 * (q % _PB) + q // _PB
    idxt = idx_phys.T
    p2 = _tc_transform(table.T, W.T, b.reshape(1, _D))
    p2v = p2.reshape(p2.shape[0] * 2, _D)
    emb3 = _sc_gather(p2v, idxt)
    y3 = _tc_unpack_first(emb3, hist, batch, batch)
    return y3.transpose(2, 0, 1)
